# baseline (speedup 1.0000x reference)
import functools

import numpy as np
import jax
import jax.numpy as jnp
from jax import lax
from jax.experimental import pallas as pl
from jax.experimental.pallas import tpu as pltpu
from jax.experimental.pallas import tpu_sc as plsc

_N = 10000
_D = 128
_E = 320000
_NLAYERS = 2
_ALPHA = 0.5
_THETA = 1.0

_NC = 2
_NS = 16
_NW = _NC * _NS
_EPT = _E // _NW
_C = 128
_NCH = 80
_EPTP = _NCH * _C
_HC = _NCH // 2
_NP = 10112
_RPT = _NP // _NS
_CW = 16

_MESH = plsc.VectorSubcoreMesh(
    core_axis_name="c", subcore_axis_name="s", num_cores=_NC, num_subcores=_NS
)


def _deg_body(col_hbm, ones_hbm, z16_hbm, out_hbm, colbuf, onesbuf, cnt_sh):
    c = lax.axis_index("c")
    s = lax.axis_index("s")
    wid = c * _NS + s
    pltpu.sync_copy(z16_hbm, cnt_sh.at[pl.ds(s * _RPT, _RPT)])
    pltpu.sync_copy(ones_hbm, onesbuf)
    pltpu.sync_copy(col_hbm.at[wid], colbuf)
    plsc.subcore_barrier()

    def chunk(j, carry):
        pltpu.sync_copy(onesbuf, cnt_sh.at[colbuf.at[j]], add=True)
        return carry

    lax.fori_loop(0, _NCH, chunk, 0)
    plsc.subcore_barrier()
    pltpu.sync_copy(
        cnt_sh.at[pl.ds(s * _RPT, _RPT)], out_hbm.at[c, pl.ds(s * _RPT, _RPT)]
    )


_deg_call = pl.kernel(
    _deg_body,
    out_type=jax.ShapeDtypeStruct((_NC, _NP, _CW), jnp.float32),
    mesh=_MESH,
    compiler_params=pltpu.CompilerParams(use_tc_tiling_on_sc=False),
    scratch_types=[
        pltpu.VMEM((_NCH, _C), jnp.int32),
        pltpu.VMEM((_C, _CW), jnp.float32),
        pltpu.VMEM_SHARED((_NP, _CW), jnp.float32),
    ],
)


def _scat_body(g_hbm, row_hbm, col_hbm, z128_hbm, out_hbm,
               rowbuf, colbuf, buf0, buf1, acc_sh, sem0, sem1):
    c = lax.axis_index("c")
    s = lax.axis_index("s")
    wid = c * _NS + s
    pltpu.sync_copy(z128_hbm, acc_sh.at[pl.ds(s * _RPT, _RPT)])
    plsc.subcore_barrier()

    for half in range(2):
        pltpu.sync_copy(row_hbm.at[wid, pl.ds(half * _HC, _HC)], rowbuf)
        pltpu.sync_copy(col_hbm.at[wid, pl.ds(half * _HC, _HC)], colbuf)

        def chunk(j, carry):
            pltpu.async_copy(g_hbm.at[rowbuf.at[j]], buf0, sem0).wait()
            pltpu.sync_copy(buf0, acc_sh.at[colbuf.at[j]], add=True)
            return carry

        lax.fori_loop(0, _HC, chunk, 0)

    plsc.subcore_barrier()
    pltpu.sync_copy(
        acc_sh.at[pl.ds(s * _RPT, _RPT)], out_hbm.at[c, pl.ds(s * _RPT, _RPT)]
    )


_scat_call = pl.kernel(
    _scat_body,
    out_type=jax.ShapeDtypeStruct((_NC, _NP, _D), jnp.float32),
    mesh=_MESH,
    scratch_types=[
        pltpu.VMEM((_HC, _C), jnp.int32),
        pltpu.VMEM((_HC, _C), jnp.int32),
        pltpu.VMEM((_C, _D), jnp.float32),
        pltpu.VMEM((_C, _D), jnp.float32),
        pltpu.VMEM_SHARED((_NP, _D), jnp.float32),
        pltpu.SemaphoreType.DMA,
        pltpu.SemaphoreType.DMA,
    ],
)


_B = 1000


def _prep_body(x_ref, cnt_ref, g0_ref):
    cnt = cnt_ref[...]
    dis = lax.rsqrt(1.0 + cnt[0, :, 0:1] + cnt[1, :, 0:1])
    g0_ref[...] = dis * jnp.maximum(x_ref[...], 0.0)


_prep_call = pl.pallas_call(
    _prep_body,
    grid=(_N // _B,),
    in_specs=[
        pl.BlockSpec((_B, _D), lambda i: (i, 0)),
        pl.BlockSpec((_NC, _B, _CW), lambda i: (0, i, 0)),
    ],
    out_specs=pl.BlockSpec((_B, _D), lambda i: (i, 0)),
    out_shape=jax.ShapeDtypeStruct((_N, _D), jnp.float32),
)


def _layer_body(s_ref, g_ref, x_ref, cnt_ref, w1_ref, w2_ref, o_ref, *,
                beta, emit_g):
    cnt = cnt_ref[...]
    dis = lax.rsqrt(1.0 + cnt[0, :, 0:1] + cnt[1, :, 0:1])
    f0 = jnp.maximum(x_ref[...], 0.0)
    h0 = _ALPHA * f0
    b = (1.0 - beta) * h0 + beta * jnp.dot(
        h0, w2_ref[0], preferred_element_type=jnp.float32
    )
    h = (1.0 - _ALPHA) * (dis * (s_ref[0] + s_ref[1] + g_ref[...]))
    out = (1.0 - beta) * h + beta * jnp.dot(
        h, w1_ref[0], preferred_element_type=jnp.float32
    ) + b
    f = jnp.maximum(out, 0.0)
    o_ref[...] = dis * f if emit_g else f


def _make_layer(li, emit_g):
    beta = float(np.log(_THETA / (li + 1) + 1.0))
    return pl.pallas_call(
        functools.partial(_layer_body, beta=beta, emit_g=emit_g),
        grid=(_N // _B,),
        in_specs=[
            pl.BlockSpec((_NC, _B, _D), lambda i: (0, i, 0)),
            pl.BlockSpec((_B, _D), lambda i: (i, 0)),
            pl.BlockSpec((_B, _D), lambda i: (i, 0)),
            pl.BlockSpec((_NC, _B, _CW), lambda i: (0, i, 0)),
            pl.BlockSpec((1, _D, _D), lambda i, _li=li: (_li, 0, 0)),
            pl.BlockSpec((1, _D, _D), lambda i, _li=li: (_li, 0, 0)),
        ],
        out_specs=pl.BlockSpec((_B, _D), lambda i: (i, 0)),
        out_shape=jax.ShapeDtypeStruct((_N, _D), jnp.float32),
    )


_layer0_call = _make_layer(0, emit_g=True)
_layer1_call = _make_layer(1, emit_g=False)


def kernel(x, edge_index, W1, W2):
    npad = _NW * _EPTP - _E
    rowp = jnp.concatenate([edge_index[0], jnp.zeros((npad,), jnp.int32)])
    colp = jnp.concatenate([edge_index[1], jnp.full((npad,), _N, jnp.int32)])
    row3 = rowp.reshape(_NW, _NCH, _C)
    col3 = colp.reshape(_NW, _NCH, _C)
    ones16 = jnp.ones((_C, _CW), jnp.float32)
    z16 = jnp.zeros((_RPT, _CW), jnp.float32)
    z128 = jnp.zeros((_RPT, _D), jnp.float32)

    cnts = _deg_call(col3, ones16, z16)
    g0 = _prep_call(x, cnts)
    s0 = _scat_call(g0, row3, col3, z128)
    g1 = _layer0_call(s0, g0, x, cnts, W1, W2)
    s1 = _scat_call(g1, row3, col3, z128)
    f2 = _layer1_call(s1, g1, x, cnts, W1, W2)
    return f2


# speedup vs baseline: 1.0182x; 1.0182x over previous
import functools

import numpy as np
import jax
import jax.numpy as jnp
from jax import lax
from jax.experimental import pallas as pl
from jax.experimental.pallas import tpu as pltpu
from jax.experimental.pallas import tpu_sc as plsc

_N = 10000
_D = 128
_E = 320000
_NLAYERS = 2
_ALPHA = 0.5
_THETA = 1.0

_NC = 2
_NS = 16
_NW = _NC * _NS
_EPT = _E // _NW
_C = 128
_NCH = 80
_EPTP = _NCH * _C
_HC = _NCH // 2
_NP = 10112
_RPT = _NP // _NS
_CW = 16

_MESH = plsc.VectorSubcoreMesh(
    core_axis_name="c", subcore_axis_name="s", num_cores=_NC, num_subcores=_NS
)


def _deg_body(col_hbm, ones_hbm, z16_hbm, out_hbm, colbuf, onesbuf, cnt_sh):
    c = lax.axis_index("c")
    s = lax.axis_index("s")
    wid = c * _NS + s
    pltpu.sync_copy(z16_hbm, cnt_sh.at[pl.ds(s * _RPT, _RPT)])
    pltpu.sync_copy(ones_hbm, onesbuf)
    pltpu.sync_copy(col_hbm.at[wid], colbuf)
    plsc.subcore_barrier()

    def chunk(j, carry):
        pltpu.sync_copy(onesbuf, cnt_sh.at[colbuf.at[j]], add=True)
        return carry

    lax.fori_loop(0, _NCH, chunk, 0)
    plsc.subcore_barrier()
    pltpu.sync_copy(
        cnt_sh.at[pl.ds(s * _RPT, _RPT)], out_hbm.at[c, pl.ds(s * _RPT, _RPT)]
    )


_deg_call = pl.kernel(
    _deg_body,
    out_type=jax.ShapeDtypeStruct((_NC, _NP, _CW), jnp.float32),
    mesh=_MESH,
    compiler_params=pltpu.CompilerParams(use_tc_tiling_on_sc=False),
    scratch_types=[
        pltpu.VMEM((_NCH, _C), jnp.int32),
        pltpu.VMEM((_C, _CW), jnp.float32),
        pltpu.VMEM_SHARED((_NP, _CW), jnp.float32),
    ],
)


def _scat_body(g_hbm, row_hbm, col_hbm, z128_hbm, out_hbm,
               rowbuf, colbuf, buf0, buf1, acc_sh, sem0, sem1):
    c = lax.axis_index("c")
    s = lax.axis_index("s")
    wid = c * _NS + s
    pltpu.sync_copy(z128_hbm, acc_sh.at[pl.ds(s * _RPT, _RPT)])
    plsc.subcore_barrier()

    for half in range(2):
        pltpu.sync_copy(row_hbm.at[wid, pl.ds(half * _HC, _HC)], rowbuf)
        pltpu.sync_copy(col_hbm.at[wid, pl.ds(half * _HC, _HC)], colbuf)

        def chunk(j, carry):
            pltpu.async_copy(g_hbm.at[rowbuf.at[j]], buf0, sem0).wait()
            pltpu.sync_copy(buf0, acc_sh.at[colbuf.at[j]], add=True)
            return carry

        lax.fori_loop(0, _HC, chunk, 0)

    plsc.subcore_barrier()
    pltpu.sync_copy(
        acc_sh.at[pl.ds(s * _RPT, _RPT)], out_hbm.at[c, pl.ds(s * _RPT, _RPT)]
    )


_scat_call = pl.kernel(
    _scat_body,
    out_type=jax.ShapeDtypeStruct((_NC, _NP, _D), jnp.float32),
    mesh=_MESH,
    scratch_types=[
        pltpu.VMEM((_HC, _C), jnp.int32),
        pltpu.VMEM((_HC, _C), jnp.int32),
        pltpu.VMEM((_C, _D), jnp.float32),
        pltpu.VMEM((_C, _D), jnp.float32),
        pltpu.VMEM_SHARED((_NP, _D), jnp.float32),
        pltpu.SemaphoreType.DMA,
        pltpu.SemaphoreType.DMA,
    ],
)


_B = 1000


def _prep_body(x_ref, cnt_ref, g0_ref):
    cnt = cnt_ref[...]
    dis = lax.rsqrt(1.0 + cnt[0, :, 0:1] + cnt[1, :, 0:1])
    g0_ref[...] = dis * jnp.maximum(x_ref[...], 0.0)


_prep_call = pl.pallas_call(
    _prep_body,
    grid=(_N // _B,),
    in_specs=[
        pl.BlockSpec((_B, _D), lambda i: (i, 0)),
        pl.BlockSpec((_NC, _B, _CW), lambda i: (0, i, 0)),
    ],
    out_specs=pl.BlockSpec((_B, _D), lambda i: (i, 0)),
    out_shape=jax.ShapeDtypeStruct((_N, _D), jnp.float32),
)


def _layer_body(s_ref, g_ref, x_ref, cnt_ref, w1_ref, w2_ref, o_ref, *,
                beta, emit_g):
    cnt = cnt_ref[...]
    dis = lax.rsqrt(1.0 + cnt[0, :, 0:1] + cnt[1, :, 0:1])
    f0 = jnp.maximum(x_ref[...], 0.0)
    h0 = _ALPHA * f0
    b = (1.0 - beta) * h0 + beta * jnp.dot(
        h0, w2_ref[0], preferred_element_type=jnp.float32
    )
    h = (1.0 - _ALPHA) * (dis * (s_ref[0] + s_ref[1] + g_ref[...]))
    out = (1.0 - beta) * h + beta * jnp.dot(
        h, w1_ref[0], preferred_element_type=jnp.float32
    ) + b
    f = jnp.maximum(out, 0.0)
    o_ref[...] = dis * f if emit_g else f


def _make_layer(li, emit_g):
    beta = float(np.log(_THETA / (li + 1) + 1.0))
    return pl.pallas_call(
        functools.partial(_layer_body, beta=beta, emit_g=emit_g),
        grid=(_N // _B,),
        in_specs=[
            pl.BlockSpec((_NC, _B, _D), lambda i: (0, i, 0)),
            pl.BlockSpec((_B, _D), lambda i: (i, 0)),
            pl.BlockSpec((_B, _D), lambda i: (i, 0)),
            pl.BlockSpec((_NC, _B, _CW), lambda i: (0, i, 0)),
            pl.BlockSpec((1, _D, _D), lambda i, _li=li: (_li, 0, 0)),
            pl.BlockSpec((1, _D, _D), lambda i, _li=li: (_li, 0, 0)),
        ],
        out_specs=pl.BlockSpec((_B, _D), lambda i: (i, 0)),
        out_shape=jax.ShapeDtypeStruct((_N, _D), jnp.float32),
    )


_layer0_call = _make_layer(0, emit_g=True)
_layer1_call = _make_layer(1, emit_g=False)


def kernel(x, edge_index, W1, W2):
    npad = _NW * _EPTP - _E
    padcol = _N + jnp.arange(npad, dtype=jnp.int32) % (_NP - _N)
    rowp = jnp.concatenate([edge_index[0], jnp.zeros((npad,), jnp.int32)])
    colp = jnp.concatenate([edge_index[1], padcol])
    row3 = rowp.reshape(_NW, _NCH, _C)
    col3 = colp.reshape(_NW, _NCH, _C)
    ones16 = jnp.ones((_C, _CW), jnp.float32)
    z16 = jnp.zeros((_RPT, _CW), jnp.float32)
    z128 = jnp.zeros((_RPT, _D), jnp.float32)

    cnts = _deg_call(col3, ones16, z16)
    g0 = _prep_call(x, cnts)
    s0 = _scat_call(g0, row3, col3, z128)
    g1 = _layer0_call(s0, g0, x, cnts, W1, W2)
    s1 = _scat_call(g1, row3, col3, z128)
    f2 = _layer1_call(s1, g1, x, cnts, W1, W2)
    return f2


# speedup vs baseline: 1.0189x; 1.0007x over previous
import functools

import numpy as np
import jax
import jax.numpy as jnp
from jax import lax
from jax.experimental import pallas as pl
from jax.experimental.pallas import tpu as pltpu
from jax.experimental.pallas import tpu_sc as plsc

_N = 10000
_D = 128
_E = 320000
_NLAYERS = 2
_ALPHA = 0.5
_THETA = 1.0

_NC = 2
_NS = 16
_NW = _NC * _NS
_EPT = _E // _NW
_C = 128
_NCH = 80
_EPTP = _NCH * _C
_HC = _NCH // 2
_NP = 10112
_RPT = _NP // _NS
_CW = 16

_MESH = plsc.VectorSubcoreMesh(
    core_axis_name="c", subcore_axis_name="s", num_cores=_NC, num_subcores=_NS
)


def _deg_body(col_hbm, ones_hbm, z16_hbm, out_hbm, colbuf, onesbuf, cnt_sh):
    c = lax.axis_index("c")
    s = lax.axis_index("s")
    wid = c * _NS + s
    pltpu.sync_copy(z16_hbm, cnt_sh.at[pl.ds(s * _RPT, _RPT)])
    pltpu.sync_copy(ones_hbm, onesbuf)
    pltpu.sync_copy(col_hbm.at[wid], colbuf)
    plsc.subcore_barrier()

    def chunk(j, carry):
        pltpu.sync_copy(onesbuf, cnt_sh.at[colbuf.at[j]], add=True)
        return carry

    lax.fori_loop(0, _NCH, chunk, 0)
    plsc.subcore_barrier()
    pltpu.sync_copy(
        cnt_sh.at[pl.ds(s * _RPT, _RPT)], out_hbm.at[c, pl.ds(s * _RPT, _RPT)]
    )


_deg_call = pl.kernel(
    _deg_body,
    out_type=jax.ShapeDtypeStruct((_NC, _NP, _CW), jnp.float32),
    mesh=_MESH,
    compiler_params=pltpu.CompilerParams(use_tc_tiling_on_sc=False),
    scratch_types=[
        pltpu.VMEM((_NCH, _C), jnp.int32),
        pltpu.VMEM((_C, _CW), jnp.float32),
        pltpu.VMEM_SHARED((_NP, _CW), jnp.float32),
    ],
)


def _scat_body(g_hbm, row_hbm, col_hbm, z128_hbm, out_hbm,
               rowbuf, colbuf, buf0, acc_sh, sem0):
    c = lax.axis_index("c")
    s = lax.axis_index("s")
    wid = c * _NS + s
    pltpu.sync_copy(z128_hbm, acc_sh.at[pl.ds(s * _RPT, _RPT)])
    plsc.subcore_barrier()

    pltpu.sync_copy(row_hbm.at[wid], rowbuf)
    pltpu.sync_copy(col_hbm.at[wid], colbuf)

    def chunk(j, carry):
        pltpu.async_copy(g_hbm.at[rowbuf.at[j]], buf0, sem0).wait()
        pltpu.sync_copy(buf0, acc_sh.at[colbuf.at[j]], add=True)
        return carry

    lax.fori_loop(0, _NCH, chunk, 0)

    plsc.subcore_barrier()
    pltpu.sync_copy(
        acc_sh.at[pl.ds(s * _RPT, _RPT)], out_hbm.at[c, pl.ds(s * _RPT, _RPT)]
    )


_scat_call = pl.kernel(
    _scat_body,
    out_type=jax.ShapeDtypeStruct((_NC, _NP, _D), jnp.float32),
    mesh=_MESH,
    scratch_types=[
        pltpu.VMEM((_NCH, _C), jnp.int32),
        pltpu.VMEM((_NCH, _C), jnp.int32),
        pltpu.VMEM((_C, _D), jnp.float32),
        pltpu.VMEM_SHARED((_NP, _D), jnp.float32),
        pltpu.SemaphoreType.DMA,
    ],
)


_B = 1000


def _prep_body(x_ref, cnt_ref, g0_ref):
    cnt = cnt_ref[...]
    dis = lax.rsqrt(1.0 + cnt[0, :, 0:1] + cnt[1, :, 0:1])
    g0_ref[...] = dis * jnp.maximum(x_ref[...], 0.0)


_prep_call = pl.pallas_call(
    _prep_body,
    grid=(_N // _B,),
    in_specs=[
        pl.BlockSpec((_B, _D), lambda i: (i, 0)),
        pl.BlockSpec((_NC, _B, _CW), lambda i: (0, i, 0)),
    ],
    out_specs=pl.BlockSpec((_B, _D), lambda i: (i, 0)),
    out_shape=jax.ShapeDtypeStruct((_N, _D), jnp.float32),
)


def _layer_body(s_ref, g_ref, x_ref, cnt_ref, w1_ref, w2_ref, o_ref, *,
                beta, emit_g):
    cnt = cnt_ref[...]
    dis = lax.rsqrt(1.0 + cnt[0, :, 0:1] + cnt[1, :, 0:1])
    f0 = jnp.maximum(x_ref[...], 0.0)
    h0 = _ALPHA * f0
    b = (1.0 - beta) * h0 + beta * jnp.dot(
        h0, w2_ref[0], preferred_element_type=jnp.float32
    )
    h = (1.0 - _ALPHA) * (dis * (s_ref[0] + s_ref[1] + g_ref[...]))
    out = (1.0 - beta) * h + beta * jnp.dot(
        h, w1_ref[0], preferred_element_type=jnp.float32
    ) + b
    f = jnp.maximum(out, 0.0)
    o_ref[...] = dis * f if emit_g else f


def _make_layer(li, emit_g):
    beta = float(np.log(_THETA / (li + 1) + 1.0))
    return pl.pallas_call(
        functools.partial(_layer_body, beta=beta, emit_g=emit_g),
        grid=(_N // _B,),
        in_specs=[
            pl.BlockSpec((_NC, _B, _D), lambda i: (0, i, 0)),
            pl.BlockSpec((_B, _D), lambda i: (i, 0)),
            pl.BlockSpec((_B, _D), lambda i: (i, 0)),
            pl.BlockSpec((_NC, _B, _CW), lambda i: (0, i, 0)),
            pl.BlockSpec((1, _D, _D), lambda i, _li=li: (_li, 0, 0)),
            pl.BlockSpec((1, _D, _D), lambda i, _li=li: (_li, 0, 0)),
        ],
        out_specs=pl.BlockSpec((_B, _D), lambda i: (i, 0)),
        out_shape=jax.ShapeDtypeStruct((_N, _D), jnp.float32),
    )


_layer0_call = _make_layer(0, emit_g=True)
_layer1_call = _make_layer(1, emit_g=False)


def kernel(x, edge_index, W1, W2):
    npad = _NW * _EPTP - _E
    padcol = _N + jnp.arange(npad, dtype=jnp.int32) % (_NP - _N)
    rowp = jnp.concatenate([edge_index[0], jnp.zeros((npad,), jnp.int32)])
    colp = jnp.concatenate([edge_index[1], padcol])
    row3 = rowp.reshape(_NW, _NCH, _C)
    col3 = colp.reshape(_NW, _NCH, _C)
    ones16 = jnp.ones((_C, _CW), jnp.float32)
    z16 = jnp.zeros((_RPT, _CW), jnp.float32)
    z128 = jnp.zeros((_RPT, _D), jnp.float32)

    cnts = _deg_call(col3, ones16, z16)
    g0 = _prep_call(x, cnts)
    s0 = _scat_call(g0, row3, col3, z128)
    g1 = _layer0_call(s0, g0, x, cnts, W1, W2)
    s1 = _scat_call(g1, row3, col3, z128)
    f2 = _layer1_call(s1, g1, x, cnts, W1, W2)
    return f2


# speedup vs baseline: 3.6109x; 3.5440x over previous
import functools

import numpy as np
import jax
import jax.numpy as jnp
from jax import lax
from jax.experimental import pallas as pl
from jax.experimental.pallas import tpu as pltpu
from jax.experimental.pallas import tpu_sc as plsc

_N = 10000
_D = 128
_E = 320000
_NLAYERS = 2
_ALPHA = 0.5
_THETA = 1.0

_NC = 2
_NS = 16
_NW = _NC * _NS
_EPT = _E // _NW
_C = 125
_NCH = _EPT // _C
_HC = _NCH // 2
_NP = 10112
_RPT = _NP // _NS
_CW = 16

_MESH = plsc.VectorSubcoreMesh(
    core_axis_name="c", subcore_axis_name="s", num_cores=_NC, num_subcores=_NS
)


def _deg_body(col_hbm, ones_hbm, z16_hbm, out_hbm, colbuf, onesbuf, cnt_sh):
    c = lax.axis_index("c")
    s = lax.axis_index("s")
    wid = c * _NS + s
    pltpu.sync_copy(z16_hbm, cnt_sh.at[pl.ds(s * _RPT, _RPT)])
    pltpu.sync_copy(ones_hbm, onesbuf)
    pltpu.sync_copy(col_hbm.at[wid], colbuf)
    plsc.subcore_barrier()

    def chunk(j, carry):
        pltpu.sync_copy(onesbuf, cnt_sh.at[colbuf.at[j]], add=True)
        return carry

    lax.fori_loop(0, _NCH, chunk, 0)
    plsc.subcore_barrier()
    pltpu.sync_copy(
        cnt_sh.at[pl.ds(s * _RPT, _RPT)], out_hbm.at[c, pl.ds(s * _RPT, _RPT)]
    )


_deg_call = pl.kernel(
    _deg_body,
    out_type=jax.ShapeDtypeStruct((_NC, _NP, _CW), jnp.float32),
    mesh=_MESH,
    compiler_params=pltpu.CompilerParams(use_tc_tiling_on_sc=False),
    scratch_types=[
        pltpu.VMEM((_NCH, _C), jnp.int32),
        pltpu.VMEM((_C, _CW), jnp.float32),
        pltpu.VMEM_SHARED((_NP, _CW), jnp.float32),
    ],
)


def _scat_body(g_hbm, row_hbm, col_hbm, z128_hbm, out_hbm,
               rowbuf, colbuf, buf0, buf1, acc_sh, sem0, sem1):
    c = lax.axis_index("c")
    s = lax.axis_index("s")
    wid = c * _NS + s
    pltpu.sync_copy(z128_hbm, acc_sh.at[pl.ds(s * _RPT, _RPT)])
    plsc.subcore_barrier()

    for half in range(2):
        pltpu.sync_copy(row_hbm.at[wid, pl.ds(half * _HC, _HC)], rowbuf)
        pltpu.sync_copy(col_hbm.at[wid, pl.ds(half * _HC, _HC)], colbuf)
        pltpu.async_copy(g_hbm.at[rowbuf.at[0]], buf0, sem0)
        pltpu.async_copy(g_hbm.at[rowbuf.at[1]], buf1, sem1)

        def pair(i, carry):
            j = 2 * i
            pltpu.make_async_copy(g_hbm.at[rowbuf.at[j]], buf0, sem0).wait()
            pltpu.sync_copy(buf0, acc_sh.at[colbuf.at[j]], add=True)
            pltpu.async_copy(g_hbm.at[rowbuf.at[j + 2]], buf0, sem0)
            pltpu.make_async_copy(g_hbm.at[rowbuf.at[j + 1]], buf1, sem1).wait()
            pltpu.sync_copy(buf1, acc_sh.at[colbuf.at[j + 1]], add=True)
            pltpu.async_copy(g_hbm.at[rowbuf.at[j + 3]], buf1, sem1)
            return carry

        lax.fori_loop(0, (_HC - 2) // 2, pair, 0)
        pltpu.make_async_copy(g_hbm.at[rowbuf.at[_HC - 2]], buf0, sem0).wait()
        pltpu.sync_copy(buf0, acc_sh.at[colbuf.at[_HC - 2]], add=True)
        pltpu.make_async_copy(g_hbm.at[rowbuf.at[_HC - 1]], buf1, sem1).wait()
        pltpu.sync_copy(buf1, acc_sh.at[colbuf.at[_HC - 1]], add=True)

    plsc.subcore_barrier()
    pltpu.sync_copy(
        acc_sh.at[pl.ds(s * _RPT, _RPT)], out_hbm.at[c, pl.ds(s * _RPT, _RPT)]
    )


_scat_call = pl.kernel(
    _scat_body,
    out_type=jax.ShapeDtypeStruct((_NC, _NP, _D), jnp.float32),
    mesh=_MESH,
    scratch_types=[
        pltpu.VMEM((_HC, _C), jnp.int32),
        pltpu.VMEM((_HC, _C), jnp.int32),
        pltpu.VMEM((_C, _D), jnp.float32),
        pltpu.VMEM((_C, _D), jnp.float32),
        pltpu.VMEM_SHARED((_NP, _D), jnp.float32),
        pltpu.SemaphoreType.DMA,
        pltpu.SemaphoreType.DMA,
    ],
)


_B = 1000


def _prep_body(x_ref, cnt_ref, g0_ref):
    cnt = cnt_ref[...]
    dis = lax.rsqrt(1.0 + cnt[0, :, 0:1] + cnt[1, :, 0:1])
    g0_ref[...] = dis * jnp.maximum(x_ref[...], 0.0)


_prep_call = pl.pallas_call(
    _prep_body,
    grid=(_N // _B,),
    in_specs=[
        pl.BlockSpec((_B, _D), lambda i: (i, 0)),
        pl.BlockSpec((_NC, _B, _CW), lambda i: (0, i, 0)),
    ],
    out_specs=pl.BlockSpec((_B, _D), lambda i: (i, 0)),
    out_shape=jax.ShapeDtypeStruct((_N, _D), jnp.float32),
)


def _layer_body(s_ref, g_ref, x_ref, cnt_ref, w1_ref, w2_ref, o_ref, *,
                beta, emit_g):
    cnt = cnt_ref[...]
    dis = lax.rsqrt(1.0 + cnt[0, :, 0:1] + cnt[1, :, 0:1])
    f0 = jnp.maximum(x_ref[...], 0.0)
    h0 = _ALPHA * f0
    b = (1.0 - beta) * h0 + beta * jnp.dot(
        h0, w2_ref[0], preferred_element_type=jnp.float32
    )
    h = (1.0 - _ALPHA) * (dis * (s_ref[0] + s_ref[1] + g_ref[...]))
    out = (1.0 - beta) * h + beta * jnp.dot(
        h, w1_ref[0], preferred_element_type=jnp.float32
    ) + b
    f = jnp.maximum(out, 0.0)
    o_ref[...] = dis * f if emit_g else f


def _make_layer(li, emit_g):
    beta = float(np.log(_THETA / (li + 1) + 1.0))
    return pl.pallas_call(
        functools.partial(_layer_body, beta=beta, emit_g=emit_g),
        grid=(_N // _B,),
        in_specs=[
            pl.BlockSpec((_NC, _B, _D), lambda i: (0, i, 0)),
            pl.BlockSpec((_B, _D), lambda i: (i, 0)),
            pl.BlockSpec((_B, _D), lambda i: (i, 0)),
            pl.BlockSpec((_NC, _B, _CW), lambda i: (0, i, 0)),
            pl.BlockSpec((1, _D, _D), lambda i, _li=li: (_li, 0, 0)),
            pl.BlockSpec((1, _D, _D), lambda i, _li=li: (_li, 0, 0)),
        ],
        out_specs=pl.BlockSpec((_B, _D), lambda i: (i, 0)),
        out_shape=jax.ShapeDtypeStruct((_N, _D), jnp.float32),
    )


_layer0_call = _make_layer(0, emit_g=True)
_layer1_call = _make_layer(1, emit_g=False)


def kernel(x, edge_index, W1, W2):
    row3 = edge_index[0].reshape(_NW, _NCH, _C)
    col3 = edge_index[1].reshape(_NW, _NCH, _C)
    ones16 = jnp.ones((_C, _CW), jnp.float32)
    z16 = jnp.zeros((_RPT, _CW), jnp.float32)
    z128 = jnp.zeros((_RPT, _D), jnp.float32)

    cnts = _deg_call(col3, ones16, z16)
    g0 = _prep_call(x, cnts)
    s0 = _scat_call(g0, row3, col3, z128)
    g1 = _layer0_call(s0, g0, x, cnts, W1, W2)
    s1 = _scat_call(g1, row3, col3, z128)
    f2 = _layer1_call(s1, g1, x, cnts, W1, W2)
    return f2


# speedup vs baseline: 3.6356x; 1.0068x over previous
import functools

import numpy as np
import jax
import jax.numpy as jnp
from jax import lax
from jax.experimental import pallas as pl
from jax.experimental.pallas import tpu as pltpu
from jax.experimental.pallas import tpu_sc as plsc

_N = 10000
_D = 128
_E = 320000
_NLAYERS = 2
_ALPHA = 0.5
_THETA = 1.0

_NC = 2
_NS = 16
_NW = _NC * _NS
_EPT = _E // _NW
_C = 125
_NCH = _EPT // _C
_HC = _NCH // 2
_NP = 10112
_RPT = _NP // _NS
_CW = 16

_MESH = plsc.VectorSubcoreMesh(
    core_axis_name="c", subcore_axis_name="s", num_cores=_NC, num_subcores=_NS
)


def _deg_body(col_hbm, ones_hbm, z16_hbm, out_hbm, colbuf, onesbuf, cnt_sh, sem):
    c = lax.axis_index("c")
    s = lax.axis_index("s")
    wid = c * _NS + s
    pltpu.sync_copy(z16_hbm, cnt_sh.at[pl.ds(s * _RPT, _RPT)])
    pltpu.sync_copy(ones_hbm, onesbuf)
    pltpu.sync_copy(col_hbm.at[wid], colbuf)
    plsc.subcore_barrier()

    def group(gi, carry):
        j = 8 * gi
        for k in range(8):
            pltpu.async_copy(onesbuf, cnt_sh.at[colbuf.at[j + k]], sem,
                             add=True)
        for k in range(8):
            pltpu.make_async_copy(onesbuf, cnt_sh.at[colbuf.at[j + k]],
                                  sem).wait()
        return carry

    lax.fori_loop(0, _NCH // 8, group, 0)
    plsc.subcore_barrier()
    pltpu.sync_copy(
        cnt_sh.at[pl.ds(s * _RPT, _RPT)], out_hbm.at[c, pl.ds(s * _RPT, _RPT)]
    )


_deg_call = pl.kernel(
    _deg_body,
    out_type=jax.ShapeDtypeStruct((_NC, _NP, _CW), jnp.float32),
    mesh=_MESH,
    compiler_params=pltpu.CompilerParams(use_tc_tiling_on_sc=False),
    scratch_types=[
        pltpu.VMEM((_NCH, _C), jnp.int32),
        pltpu.VMEM((_C, _CW), jnp.float32),
        pltpu.VMEM_SHARED((_NP, _CW), jnp.float32),
        pltpu.SemaphoreType.DMA,
    ],
)


def _scat_body(g_hbm, row_hbm, col_hbm, z128_hbm, out_hbm,
               rowbuf, colbuf, buf0, buf1, acc_sh, sem0, sem1):
    c = lax.axis_index("c")
    s = lax.axis_index("s")
    wid = c * _NS + s
    pltpu.sync_copy(z128_hbm, acc_sh.at[pl.ds(s * _RPT, _RPT)])
    plsc.subcore_barrier()

    for half in range(2):
        pltpu.sync_copy(row_hbm.at[wid, pl.ds(half * _HC, _HC)], rowbuf)
        pltpu.sync_copy(col_hbm.at[wid, pl.ds(half * _HC, _HC)], colbuf)
        pltpu.async_copy(g_hbm.at[rowbuf.at[0]], buf0, sem0)
        pltpu.async_copy(g_hbm.at[rowbuf.at[1]], buf1, sem1)

        def pair(i, carry):
            j = 2 * i
            pltpu.make_async_copy(g_hbm.at[rowbuf.at[j]], buf0, sem0).wait()
            pltpu.sync_copy(buf0, acc_sh.at[colbuf.at[j]], add=True)
            pltpu.async_copy(g_hbm.at[rowbuf.at[j + 2]], buf0, sem0)
            pltpu.make_async_copy(g_hbm.at[rowbuf.at[j + 1]], buf1, sem1).wait()
            pltpu.sync_copy(buf1, acc_sh.at[colbuf.at[j + 1]], add=True)
            pltpu.async_copy(g_hbm.at[rowbuf.at[j + 3]], buf1, sem1)
            return carry

        lax.fori_loop(0, (_HC - 2) // 2, pair, 0)
        pltpu.make_async_copy(g_hbm.at[rowbuf.at[_HC - 2]], buf0, sem0).wait()
        pltpu.sync_copy(buf0, acc_sh.at[colbuf.at[_HC - 2]], add=True)
        pltpu.make_async_copy(g_hbm.at[rowbuf.at[_HC - 1]], buf1, sem1).wait()
        pltpu.sync_copy(buf1, acc_sh.at[colbuf.at[_HC - 1]], add=True)

    plsc.subcore_barrier()
    pltpu.sync_copy(
        acc_sh.at[pl.ds(s * _RPT, _RPT)], out_hbm.at[c, pl.ds(s * _RPT, _RPT)]
    )


_scat_call = pl.kernel(
    _scat_body,
    out_type=jax.ShapeDtypeStruct((_NC, _NP, _D), jnp.float32),
    mesh=_MESH,
    scratch_types=[
        pltpu.VMEM((_HC, _C), jnp.int32),
        pltpu.VMEM((_HC, _C), jnp.int32),
        pltpu.VMEM((_C, _D), jnp.float32),
        pltpu.VMEM((_C, _D), jnp.float32),
        pltpu.VMEM_SHARED((_NP, _D), jnp.float32),
        pltpu.SemaphoreType.DMA,
        pltpu.SemaphoreType.DMA,
    ],
)


_B = 1000


def _prep_body(x_ref, cnt_ref, g0_ref):
    cnt = cnt_ref[...]
    dis = lax.rsqrt(1.0 + cnt[0, :, 0:1] + cnt[1, :, 0:1])
    g0_ref[...] = dis * jnp.maximum(x_ref[...], 0.0)


_prep_call = pl.pallas_call(
    _prep_body,
    grid=(_N // _B,),
    in_specs=[
        pl.BlockSpec((_B, _D), lambda i: (i, 0)),
        pl.BlockSpec((_NC, _B, _CW), lambda i: (0, i, 0)),
    ],
    out_specs=pl.BlockSpec((_B, _D), lambda i: (i, 0)),
    out_shape=jax.ShapeDtypeStruct((_N, _D), jnp.float32),
)


_BETA0 = float(np.log(_THETA / 1.0 + 1.0))
_BETA1 = float(np.log(_THETA / 2.0 + 1.0))


def _bterm_body(x_ref, w2_ref, b0_ref, b1_ref):
    h0 = _ALPHA * jnp.maximum(x_ref[...], 0.0)
    b0_ref[...] = (1.0 - _BETA0) * h0 + _BETA0 * jnp.dot(
        h0, w2_ref[0], preferred_element_type=jnp.float32)
    b1_ref[...] = (1.0 - _BETA1) * h0 + _BETA1 * jnp.dot(
        h0, w2_ref[1], preferred_element_type=jnp.float32)


_bterm_call = pl.pallas_call(
    _bterm_body,
    grid=(_N // _B,),
    in_specs=[
        pl.BlockSpec((_B, _D), lambda i: (i, 0)),
        pl.BlockSpec((_NLAYERS, _D, _D), lambda i: (0, 0, 0)),
    ],
    out_specs=[
        pl.BlockSpec((_B, _D), lambda i: (i, 0)),
        pl.BlockSpec((_B, _D), lambda i: (i, 0)),
    ],
    out_shape=[
        jax.ShapeDtypeStruct((_N, _D), jnp.float32),
        jax.ShapeDtypeStruct((_N, _D), jnp.float32),
    ],
)


def _layer_body(s_ref, g_ref, b_ref, cnt_ref, w1_ref, o_ref, *,
                beta, emit_g):
    cnt = cnt_ref[...]
    dis = lax.rsqrt(1.0 + cnt[0, :, 0:1] + cnt[1, :, 0:1])
    h = (1.0 - _ALPHA) * (dis * (s_ref[0] + s_ref[1] + g_ref[...]))
    out = (1.0 - beta) * h + beta * jnp.dot(
        h, w1_ref[0], preferred_element_type=jnp.float32
    ) + b_ref[...]
    f = jnp.maximum(out, 0.0)
    o_ref[...] = dis * f if emit_g else f


def _make_layer(li, emit_g):
    beta = float(np.log(_THETA / (li + 1) + 1.0))
    return pl.pallas_call(
        functools.partial(_layer_body, beta=beta, emit_g=emit_g),
        grid=(_N // _B,),
        in_specs=[
            pl.BlockSpec((_NC, _B, _D), lambda i: (0, i, 0)),
            pl.BlockSpec((_B, _D), lambda i: (i, 0)),
            pl.BlockSpec((_B, _D), lambda i: (i, 0)),
            pl.BlockSpec((_NC, _B, _CW), lambda i: (0, i, 0)),
            pl.BlockSpec((1, _D, _D), lambda i, _li=li: (_li, 0, 0)),
        ],
        out_specs=pl.BlockSpec((_B, _D), lambda i: (i, 0)),
        out_shape=jax.ShapeDtypeStruct((_N, _D), jnp.float32),
    )


_layer0_call = _make_layer(0, emit_g=True)
_layer1_call = _make_layer(1, emit_g=False)


def kernel(x, edge_index, W1, W2):
    row3 = edge_index[0].reshape(_NW, _NCH, _C)
    col3 = edge_index[1].reshape(_NW, _NCH, _C)
    ones16 = jnp.ones((_C, _CW), jnp.float32)
    z16 = jnp.zeros((_RPT, _CW), jnp.float32)
    z128 = jnp.zeros((_RPT, _D), jnp.float32)

    cnts = _deg_call(col3, ones16, z16)
    g0 = _prep_call(x, cnts)
    b0, b1 = _bterm_call(x, W2)
    s0 = _scat_call(g0, row3, col3, z128)
    g1 = _layer0_call(s0, g0, b0, cnts, W1)
    s1 = _scat_call(g1, row3, col3, z128)
    f2 = _layer1_call(s1, g1, b1, cnts, W1)
    return f2
